# initial kernel scaffold (unmeasured)
import jax
import jax.numpy as jnp
from jax import lax
from jax.experimental import pallas as pl
from jax.experimental.pallas import tpu as pltpu

N = 32
M = 1024
D = 1024
ROWS = M // N


def kernel(partial, resid, gamma):

    def body(x_ref, resid_ref, gamma_ref, out_ref,
             rs_recv, rs_send,
             rs_send_sems, rs_recv_sems, ag_send_sems, ag_recv_sems):
        my = lax.axis_index("i")
        right = (my + 1) % N

        def rows(cid):
            return pl.ds(cid * ROWS, ROWS)

        for h in range(N - 1):
            sc = (my - h) % N
            if h == 0:
                src = x_ref.at[0, rows(sc), :]
            else:
                rs_send[h] = rs_recv[h - 1] + x_ref[0, rows(sc), :]
                src = rs_send.at[h]
            rdma = pltpu.make_async_remote_copy(
                src_ref=src,
                dst_ref=rs_recv.at[h],
                send_sem=rs_send_sems.at[h],
                recv_sem=rs_recv_sems.at[h],
                device_id=(right,),
                device_id_type=pl.DeviceIdType.MESH,
            )
            rdma.start()
            rdma.wait()

        myc = (my + 1) % N
        y = rs_recv[N - 2] + x_ref[0, rows(myc), :] + resid_ref[rows(myc), :]
        rms = jnp.sqrt(jnp.mean(y * y, axis=-1, keepdims=True) + 1e-6)
        out_ref[rows(myc), :] = y / rms * gamma_ref[:][None, :]

        for h in range(N - 1):
            s_id = (my + 1 - h) % N
            rdma = pltpu.make_async_remote_copy(
                src_ref=out_ref.at[rows(s_id), :],
                dst_ref=out_ref.at[rows(s_id), :],
                send_sem=ag_send_sems.at[h],
                recv_sem=ag_recv_sems.at[h],
                device_id=(right,),
                device_id_type=pl.DeviceIdType.MESH,
            )
            rdma.start()
            rdma.wait()

    return pl.pallas_call(
        body,
        out_shape=jax.ShapeDtypeStruct((M, D), jnp.float32),
        in_specs=[
            pl.BlockSpec(memory_space=pltpu.VMEM),
            pl.BlockSpec(memory_space=pltpu.VMEM),
            pl.BlockSpec(memory_space=pltpu.VMEM),
        ],
        out_specs=pl.BlockSpec(memory_space=pltpu.VMEM),
        scratch_shapes=[
            pltpu.VMEM((N - 1, ROWS, D), jnp.float32),
            pltpu.VMEM((N - 1, ROWS, D), jnp.float32),
            pltpu.SemaphoreType.DMA((N - 1,)),
            pltpu.SemaphoreType.DMA((N - 1,)),
            pltpu.SemaphoreType.DMA((N - 1,)),
            pltpu.SemaphoreType.DMA((N - 1,)),
        ],
        compiler_params=pltpu.CompilerParams(collective_id=0),
    )(partial, resid, gamma)


# baseline (device time: 222092 ns/iter reference)
import jax
import jax.numpy as jnp
from jax import lax
from jax.experimental import pallas as pl
from jax.experimental.pallas import tpu as pltpu

N = 32
M = 1024
D = 1024
ROWS = M // N


def kernel(partial, resid, gamma):

    def body(x_ref, resid_ref, gamma_ref, out_ref,
             rs_recv, rs_send,
             rs_send_sems, rs_recv_sems, ag_send_sems, ag_recv_sems):
        my = lax.axis_index("i")
        right = (my + 1) % N

        def rows(cid):
            return pl.ds(cid * ROWS, ROWS)

        for h in range(N - 1):
            sc = (my - h) % N
            if h == 0:
                src = x_ref.at[0, rows(sc), :]
            else:
                rs_send[h] = rs_recv[h - 1] + x_ref[0, rows(sc), :]
                src = rs_send.at[h]
            rdma = pltpu.make_async_remote_copy(
                src_ref=src,
                dst_ref=rs_recv.at[h],
                send_sem=rs_send_sems.at[h],
                recv_sem=rs_recv_sems.at[h],
                device_id=(right,),
                device_id_type=pl.DeviceIdType.MESH,
            )
            rdma.start()
            rdma.wait()

        myc = (my + 1) % N
        y = rs_recv[N - 2] + x_ref[0, rows(myc), :] + resid_ref[rows(myc), :]
        rms = jnp.sqrt(jnp.mean(y * y, axis=-1, keepdims=True) + 1e-6)
        out_ref[rows(myc), :] = y / rms * gamma_ref[:][None, :]

        for h in range(N - 1):
            s_id = (my + 1 - h) % N
            rdma = pltpu.make_async_remote_copy(
                src_ref=out_ref.at[rows(s_id), :],
                dst_ref=out_ref.at[rows(s_id), :],
                send_sem=ag_send_sems.at[h],
                recv_sem=ag_recv_sems.at[h],
                device_id=(right,),
                device_id_type=pl.DeviceIdType.MESH,
            )
            rdma.start()
            rdma.wait()

    return pl.pallas_call(
        body,
        out_shape=jax.ShapeDtypeStruct((M, D), jnp.float32),
        in_specs=[
            pl.BlockSpec(memory_space=pltpu.VMEM),
            pl.BlockSpec(memory_space=pltpu.VMEM),
            pl.BlockSpec(memory_space=pltpu.VMEM),
        ],
        out_specs=pl.BlockSpec(memory_space=pltpu.VMEM),
        scratch_shapes=[
            pltpu.VMEM((N - 1, ROWS, D), jnp.float32),
            pltpu.VMEM((N - 1, ROWS, D), jnp.float32),
            pltpu.SemaphoreType.DMA((N - 1,)),
            pltpu.SemaphoreType.DMA((N - 1,)),
            pltpu.SemaphoreType.DMA((N - 1,)),
            pltpu.SemaphoreType.DMA((N - 1,)),
        ],
    )(partial, resid, gamma)
